# baseline (speedup 1.0000x reference)
import math

import jax
import jax.numpy as jnp
from jax.experimental import pallas as pl
from jax.experimental.pallas import tpu as pltpu

_V = 100000
_PAD_IDX = 0
_SMOOTH = 0.1
_CONF = 1.0 - _SMOOTH
_EPS = _SMOOTH / (_V - 2)
_ENT = (_V - 1) * _EPS * math.log(_EPS) + _CONF * math.log(_CONF)

_RB = 8
_NS = 2


def _wsum(x, tgt, valid):
    cols = jax.lax.broadcasted_iota(jnp.int32, x.shape, 1)
    hit = cols == tgt
    w = jnp.where(hit, valid * jnp.float32(_CONF), valid * jnp.float32(_EPS))
    return jnp.sum(w * x) - jnp.float32(_ENT) * jnp.sum(valid)


def _loss_kernel(*refs):
    out_ref = refs[-1]
    j = pl.program_id(0)

    @pl.when(j == 0)
    def _init():
        out_ref[0, 0] = 0.0

    acc = 0.0
    for k in range(_NS):
        t_ref, v_ref, x_ref = refs[2 * k], refs[2 * k + 1], refs[2 * _NS + k]
        acc += _wsum(x_ref[:, :], t_ref[:, :], v_ref[:, :])
    out_ref[0, 0] -= acc


def kernel(x, target):
    B, S, V = x.shape
    R = B * S
    steps = (R // _NS) // _RB
    x2 = x.reshape(R, V)
    tgt = target.astype(jnp.int32).reshape(R, 1)
    s_idx = jax.lax.broadcasted_iota(jnp.int32, (B, S), 1).reshape(R, 1)
    valid = ((tgt != _PAD_IDX) & (s_idx != _PAD_IDX)).astype(jnp.float32)
    row_specs, x_specs, row_ops, x_ops = [], [], [], []
    for k in range(_NS):
        imap = (lambda kk: (lambda j: (j + kk * steps, 0)))(k)
        row_specs += [pl.BlockSpec((_RB, 1), imap)] * 2
        x_specs.append(pl.BlockSpec((_RB, V), imap))
        row_ops += [tgt, valid]
        x_ops.append(x2)
    out = pl.pallas_call(
        _loss_kernel,
        grid=(steps,),
        in_specs=row_specs + x_specs,
        out_specs=pl.BlockSpec((1, 1), lambda j: (0, 0),
                               memory_space=pltpu.SMEM),
        out_shape=jax.ShapeDtypeStruct((1, 1), jnp.float32),
    )(*row_ops, *x_ops)
    return out[0, 0]


# speedup vs baseline: 1.1541x; 1.1541x over previous
import math

import jax
import jax.numpy as jnp
from jax.experimental import pallas as pl
from jax.experimental.pallas import tpu as pltpu

_V = 100000
_PAD_IDX = 0
_SMOOTH = 0.1
_CONF = 1.0 - _SMOOTH
_EPS = _SMOOTH / (_V - 2)
_ENT = (_V - 1) * _EPS * math.log(_EPS) + _CONF * math.log(_CONF)

_RB = 8
_NS = 4


def _wsum(x, tgt, row0, seq_len):
    rows = row0 + jax.lax.broadcasted_iota(jnp.int32, (x.shape[0], 1), 0)
    s_pos = rows - (rows // seq_len) * seq_len
    valid = ((s_pos != _PAD_IDX) & (tgt != _PAD_IDX)).astype(jnp.float32)
    cols = jax.lax.broadcasted_iota(jnp.int32, x.shape, 1)
    hit = cols == tgt
    w = jnp.where(hit, valid * jnp.float32(_CONF), valid * jnp.float32(_EPS))
    return jnp.sum(w * x) - jnp.float32(_ENT) * jnp.sum(valid)


def _loss_kernel(seq_len, steps, *refs):
    out_ref = refs[-1]
    j = pl.program_id(0)

    @pl.when(j == 0)
    def _init():
        out_ref[0, 0] = 0.0

    acc = 0.0
    for k in range(_NS):
        t_ref, x_ref = refs[k], refs[_NS + k]
        row0 = (k * steps + j) * _RB
        acc += _wsum(x_ref[:, :], t_ref[:, :], row0, seq_len)
    out_ref[0, 0] -= acc


def kernel(x, target):
    B, S, V = x.shape
    R = B * S
    steps = (R // _NS) // _RB
    x2 = x.reshape(R, V)
    tgt = target.astype(jnp.int32).reshape(R, 1)
    row_specs, x_specs, row_ops, x_ops = [], [], [], []
    for k in range(_NS):
        imap = (lambda kk: (lambda j: (j + kk * steps, 0)))(k)
        row_specs.append(pl.BlockSpec((_RB, 1), imap))
        x_specs.append(pl.BlockSpec((_RB, V), imap))
        row_ops.append(tgt)
        x_ops.append(x2)
    import functools
    out = pl.pallas_call(
        functools.partial(_loss_kernel, S, steps),
        grid=(steps,),
        in_specs=row_specs + x_specs,
        out_specs=pl.BlockSpec((1, 1), lambda j: (0, 0),
                               memory_space=pltpu.SMEM),
        out_shape=jax.ShapeDtypeStruct((1, 1), jnp.float32),
    )(*row_ops, *x_ops)
    return out[0, 0]
